# baseline (speedup 1.0000x reference)
import functools

import jax
import jax.numpy as jnp
from jax import lax
from jax.experimental import pallas as pl
from jax.experimental.pallas import tpu as pltpu
from jax.experimental.pallas import tpu_sc as plsc

B = 16384
C = 1000
MARGIN = 1.0

NC = 2
NS = 16
L = 16
NW = NC * NS

RW = B // NW
CH = 32
NCH = RW // CH
NB = C // L
TAIL = C - NB * L
TSTART = NB * L
CPAD = 1024


def _sc_hinge_body(x_hbm, y_hbm, out_hbm, ybuf, xbuf0, xbuf1, accbuf,
                   sem0, sem1):
    wid = lax.axis_index("s") * NC + lax.axis_index("c")
    base = wid * RW
    pltpu.sync_copy(y_hbm.at[pl.ds(base, RW)], ybuf)
    xbufs = (xbuf0, xbuf1)
    sems = (sem0, sem1)
    handles = {}
    for g in range(2):
        handles[g] = pltpu.async_copy(
            x_hbm.at[pl.ds(base + g * CH, CH), :], xbufs[g % 2],
            sems[g % 2])
    iota16 = lax.iota(jnp.int32, L)
    tailmask = iota16 < TAIL
    tailcols = jnp.where(tailmask, TSTART + iota16, -1)
    dnums = lax.GatherDimensionNumbers(offset_dims=(),
                                       collapsed_slice_dims=(0,),
                                       start_index_map=(0,))

    def _splat(vec, lanes):
        return lax.gather(vec, lanes[:, None], dnums, slice_sizes=(1,),
                          mode=lax.GatherScatterMode.PROMISE_IN_BOUNDS)

    def _chunk(g, acc_, xb):
        def row_body(r, a, xb=xb):
            gstart = pl.multiple_of(g * CH + (r & -L), L)
            ygrp = ybuf[pl.ds(gstart, L)]
            yr_vec = _splat(ygrp, jnp.full((L,), r & (L - 1), jnp.int32))
            oyv = jnp.zeros((L,), jnp.float32)
            for j in range(NB):
                v = xb[r, pl.ds(j * L, L)]
                oyv = oyv + jnp.where(iota16 == yr_vec - j * L, v, 0.0)
            tstart = pl.multiple_of(jnp.int32(TSTART), L)
            vt = xb[r, pl.ds(tstart, L)]
            oyv = oyv + jnp.where(tailcols == yr_vec, vt, 0.0)
            lanesel = jnp.where(yr_vec >= NB * L, yr_vec - TSTART,
                                yr_vec & (L - 1))
            ym = _splat(oyv, lanesel) - MARGIN
            for j in range(NB):
                v = xb[r, pl.ds(j * L, L)]
                a = a + jnp.maximum(v - ym, 0.0)
            vt2 = xb[r, pl.ds(tstart, L)]
            a = a + jnp.where(tailmask, jnp.maximum(vt2 - ym, 0.0), 0.0)
            return a

        return lax.fori_loop(0, CH, row_body, acc_)

    def pair_body(p, acc_):
        g0 = 2 * p
        for k, (xb, sem) in enumerate(((xbuf0, sem0), (xbuf1, sem1))):
            g = g0 + k
            pltpu.make_async_copy(
                x_hbm.at[pl.ds(base + g * CH, CH), :], xb, sem).wait()
            acc_ = _chunk(g, acc_, xb)

            @pl.when(p + 1 < NCH // 2)
            def _prefetch(g=g, xb=xb, sem=sem):
                pltpu.async_copy(
                    x_hbm.at[pl.ds(base + (g + 2) * CH, CH), :], xb, sem)
        return acc_

    acc = lax.fori_loop(0, NCH // 2, pair_body, jnp.zeros((L,), jnp.float32))
    accbuf[...] = acc
    pltpu.sync_copy(accbuf, out_hbm.at[pl.ds(wid * L, L)])


@functools.cache
def _sc_hinge():
    return pl.kernel(
        _sc_hinge_body,
        out_type=jax.ShapeDtypeStruct((NW * L,), jnp.float32),
        mesh=plsc.VectorSubcoreMesh(core_axis_name="c", subcore_axis_name="s",
                                    num_cores=NC, num_subcores=NS),
        scratch_types=[
            pltpu.VMEM((RW,), jnp.int32),
            pltpu.VMEM((CH, C), jnp.float32),
            pltpu.VMEM((CH, C), jnp.float32),
            pltpu.VMEM((L,), jnp.float32),
            pltpu.SemaphoreType.DMA,
            pltpu.SemaphoreType.DMA,
        ],
    )


def kernel(output, y):
    y32 = y.astype(jnp.int32)
    partials = _sc_hinge()(output, y32)
    return (jnp.sum(partials) - float(B)) / float(B)
